# baseline (speedup 1.0000x reference)
import functools

import jax
import jax.numpy as jnp
from jax import lax
from jax.experimental import pallas as pl
from jax.experimental.pallas import tpu as pltpu
from jax.experimental.pallas import tpu_sc as plsc

BATCH = 16384
D = 64
NUM_CORES = 2
NUM_SUBCORES = 16
NW = NUM_CORES * NUM_SUBCORES
B_PER_W = BATCH // NW
C = 128
NCHUNK = B_PER_W // C


def _lane_perm(x, idx):
    dnums = lax.GatherDimensionNumbers(
        offset_dims=(), collapsed_slice_dims=(0,), start_index_map=(0,))
    return lax.gather(x, idx[:, None], dnums, slice_sizes=(1,),
                      mode=lax.GatherScatterMode.PROMISE_IN_BOUNDS)


def _body(heads_hbm, rels_hbm, tails_hbm, eh_hbm, et_hbm, r_hbm, ri_hbm,
          out_hbm, heads_v, rels_v, tails_v,
          hh_v, ht_v, th_v, tt_v, r_v, ri_v, out_v, sem):
    cid = lax.axis_index("c")
    sid = lax.axis_index("s")
    wid = sid * NUM_CORES + cid
    base = wid * B_PER_W

    iota = lax.iota(jnp.int32, 16)

    for c in range(NCHUNK):
        off = base + c * C
        pltpu.sync_copy(heads_hbm.at[pl.ds(off, C)], heads_v)
        pltpu.sync_copy(rels_hbm.at[pl.ds(off, C)], rels_v)
        pltpu.sync_copy(tails_hbm.at[pl.ds(off, C)], tails_v)

        cps = [
            pltpu.async_copy(eh_hbm.at[heads_v], hh_v, sem),
            pltpu.async_copy(eh_hbm.at[tails_v], ht_v, sem),
            pltpu.async_copy(et_hbm.at[heads_v], th_v, sem),
            pltpu.async_copy(et_hbm.at[tails_v], tt_v, sem),
            pltpu.async_copy(r_hbm.at[rels_v], r_v, sem),
            pltpu.async_copy(ri_hbm.at[rels_v], ri_v, sem),
        ]
        for cp in cps:
            cp.wait()

        def group_body(g, _):
            def row_body(j, acc):
                row = g * 16 + j
                s = None
                for q in range(D // 16):
                    sl = pl.ds(q * 16, 16)
                    p1 = hh_v[row, sl] * r_v[row, sl] * tt_v[row, sl]
                    p2 = ht_v[row, sl] * ri_v[row, sl] * th_v[row, sl]
                    ps = p1 + p2
                    s = ps if s is None else s + ps
                for sh in (8, 4, 2, 1):
                    s = s + _lane_perm(s, iota ^ sh)
                return jnp.where(iota == j, s, acc)

            acc = lax.fori_loop(0, 16, row_body, jnp.zeros((16,), jnp.float32))
            acc = jnp.clip(acc * 0.5, -20.0, 20.0)
            off = pl.multiple_of(c * C + g * 16, 16)
            out_v[pl.ds(off, 16)] = acc
            return 0

        lax.fori_loop(0, C // 16, group_body, 0)

    pltpu.sync_copy(out_v, out_hbm.at[pl.ds(base, B_PER_W)])


def kernel(batch, ent_h_embs, ent_t_embs, rel_embs, rel_inv_embs):
    mesh = plsc.VectorSubcoreMesh(core_axis_name="c", subcore_axis_name="s")
    k = functools.partial(
        pl.kernel,
        mesh=mesh,
        compiler_params=pltpu.CompilerParams(use_tc_tiling_on_sc=False),
        out_type=jax.ShapeDtypeStruct((BATCH,), jnp.float32),
        scratch_types=[
            pltpu.VMEM((C,), jnp.int32),
            pltpu.VMEM((C,), jnp.int32),
            pltpu.VMEM((C,), jnp.int32),
            pltpu.VMEM((C, D), jnp.float32),
            pltpu.VMEM((C, D), jnp.float32),
            pltpu.VMEM((C, D), jnp.float32),
            pltpu.VMEM((C, D), jnp.float32),
            pltpu.VMEM((C, D), jnp.float32),
            pltpu.VMEM((C, D), jnp.float32),
            pltpu.VMEM((B_PER_W,), jnp.float32),
            pltpu.SemaphoreType.DMA,
        ],
    )(_body)
    heads = batch[:, 0]
    rels = batch[:, 1]
    tails = batch[:, 2]
    return k(heads, rels, tails, ent_h_embs, ent_t_embs, rel_embs, rel_inv_embs)


# speedup vs baseline: 1.4733x; 1.4733x over previous
import functools

import jax
import jax.numpy as jnp
from jax import lax
from jax.experimental import pallas as pl
from jax.experimental.pallas import tpu as pltpu
from jax.experimental.pallas import tpu_sc as plsc

BATCH = 16384
D = 64
NUM_CORES = 2
NUM_SUBCORES = 16
NW = NUM_CORES * NUM_SUBCORES
B_PER_W = BATCH // NW
W = 16
NCHUNK = B_PER_W // W


def _lane_perm(x, idx):
    dnums = lax.GatherDimensionNumbers(
        offset_dims=(), collapsed_slice_dims=(0,), start_index_map=(0,))
    return lax.gather(x, idx[:, None], dnums, slice_sizes=(1,),
                      mode=lax.GatherScatterMode.PROMISE_IN_BOUNDS)


def _body(heads_hbm, rels_hbm, tails_hbm, eh_hbm, et_hbm, r_hbm, ri_hbm,
          out_hbm, h_idx, r_idx, t_idx,
          bufs_a, bufs_b, out_v, sem_a, sem_b):
    cid = lax.axis_index("c")
    sid = lax.axis_index("s")
    wid = sid * NUM_CORES + cid
    base = wid * B_PER_W

    pltpu.sync_copy(heads_hbm.at[pl.ds(base, B_PER_W)], h_idx)
    pltpu.sync_copy(rels_hbm.at[pl.ds(base, B_PER_W)], r_idx)
    pltpu.sync_copy(tails_hbm.at[pl.ds(base, B_PER_W)], t_idx)

    iota = lax.iota(jnp.int32, 16)

    def issue(chunk, bufs, sem):
        hh_v, ht_v, th_v, tt_v, r_v, ri_v = bufs
        off = pl.multiple_of(chunk * W, W)
        hvec = h_idx[pl.ds(off, W)]
        rvec = r_idx[pl.ds(off, W)]
        tvec = t_idx[pl.ds(off, W)]
        for j in range(W):
            hv = hvec[j]
            rv = rvec[j]
            tv = tvec[j]
            pltpu.async_copy(eh_hbm.at[hv], hh_v.at[j], sem)
            pltpu.async_copy(eh_hbm.at[tv], ht_v.at[j], sem)
            pltpu.async_copy(et_hbm.at[hv], th_v.at[j], sem)
            pltpu.async_copy(et_hbm.at[tv], tt_v.at[j], sem)
            pltpu.async_copy(r_hbm.at[rv], r_v.at[j], sem)
            pltpu.async_copy(ri_hbm.at[rv], ri_v.at[j], sem)

    def drain(bufs, sem):
        for buf in bufs:
            pltpu.make_async_copy(eh_hbm.at[pl.ds(0, W)], buf, sem).wait()

    def compute(chunk, bufs):
        hh_v, ht_v, th_v, tt_v, r_v, ri_v = bufs
        acc = jnp.zeros((16,), jnp.float32)
        for j in range(W):
            s = None
            for q in range(D // 16):
                sl = pl.ds(q * 16, 16)
                p = (hh_v[j, sl] * r_v[j, sl] * tt_v[j, sl]
                     + ht_v[j, sl] * ri_v[j, sl] * th_v[j, sl])
                s = p if s is None else s + p
            for sh in (8, 4, 2, 1):
                s = s + _lane_perm(s, iota ^ sh)
            acc = jnp.where(iota == j, s, acc)
        acc = jnp.clip(acc * 0.5, -20.0, 20.0)
        out_v[pl.ds(pl.multiple_of(chunk * W, W), W)] = acc

    issue(0, bufs_a, sem_a)

    def pair_body(k, _):
        c0 = k * 2
        issue(c0 + 1, bufs_b, sem_b)
        drain(bufs_a, sem_a)
        compute(c0, bufs_a)

        @pl.when(c0 + 2 < NCHUNK)
        def _():
            issue(c0 + 2, bufs_a, sem_a)

        drain(bufs_b, sem_b)
        compute(c0 + 1, bufs_b)
        return 0

    lax.fori_loop(0, NCHUNK // 2, pair_body, 0)

    pltpu.sync_copy(out_v, out_hbm.at[pl.ds(base, B_PER_W)])


def kernel(batch, ent_h_embs, ent_t_embs, rel_embs, rel_inv_embs):
    mesh = plsc.VectorSubcoreMesh(core_axis_name="c", subcore_axis_name="s")
    row_bufs = [pltpu.VMEM((W, D), jnp.float32) for _ in range(6)]
    k = functools.partial(
        pl.kernel,
        mesh=mesh,
        out_type=jax.ShapeDtypeStruct((BATCH,), jnp.float32),
        scratch_types=[
            pltpu.VMEM((B_PER_W,), jnp.int32),
            pltpu.VMEM((B_PER_W,), jnp.int32),
            pltpu.VMEM((B_PER_W,), jnp.int32),
            row_bufs,
            [pltpu.VMEM((W, D), jnp.float32) for _ in range(6)],
            pltpu.VMEM((B_PER_W,), jnp.float32),
            pltpu.SemaphoreType.DMA,
            pltpu.SemaphoreType.DMA,
        ],
    )(_body)
    heads = batch[:, 0]
    rels = batch[:, 1]
    tails = batch[:, 2]
    return k(heads, rels, tails, ent_h_embs, ent_t_embs, rel_embs, rel_inv_embs)
